# baseline (speedup 1.0000x reference)
import functools

import jax
import jax.numpy as jnp
from jax import lax
from jax.experimental import pallas as pl
from jax.experimental.pallas import tpu as pltpu
from jax.experimental.pallas import tpu_sc as plsc

N = 10000
E = 160000
NC, NS = 2, 16
NW = NC * NS
C = 128
CHT = 40
E_PAD = NW * CHT * C
ECH = E_PAD // C
NSEM = 8

N_PAD = 10016
RPT = N_PAD // NS
ZR = 313

ND_PAD = 10240
RPTD = ND_PAD // NS

_mesh = lambda: plsc.VectorSubcoreMesh(core_axis_name="c", subcore_axis_name="s")
_SC_PARAMS = pltpu.CompilerParams(use_tc_tiling_on_sc=False)


@functools.partial(
    pl.kernel,
    out_type=jax.ShapeDtypeStruct((NC * ND_PAD,), jnp.float32),
    mesh=_mesh(),
    scratch_types=[
        pltpu.VMEM_SHARED((ND_PAD,), jnp.float32),
        pltpu.VMEM((CHT, C), jnp.int32),
        pltpu.VMEM((C,), jnp.float32),
        pltpu.VMEM((RPTD,), jnp.float32),
    ] + [pltpu.SemaphoreType.DMA] * NSEM,
    compiler_params=_SC_PARAMS,
)
def _deg(dst_hbm, out_hbm, acc, didx, ones_v, zbuf, *sems):
    cid = lax.axis_index("c")
    sid = lax.axis_index("s")
    for i in range(C // 16):
        ones_v[pl.ds(i * 16, 16)] = jnp.full((16,), 1.0, jnp.float32)
    for i in range(RPTD // 16):
        zbuf[pl.ds(i * 16, 16)] = jnp.zeros((16,), jnp.float32)
    gwid = cid * NS + sid
    pltpu.sync_copy(dst_hbm.at[pl.ds(gwid * CHT, CHT)], didx)
    pltpu.sync_copy(zbuf, acc.at[pl.ds(sid * RPTD, RPTD)])
    plsc.subcore_barrier()

    @pl.loop(0, CHT // NSEM)
    def _(r):
        c0 = r * NSEM
        descs = [
            pltpu.async_copy(ones_v, acc.at[didx.at[c0 + b]], sems[b], add=True)
            for b in range(NSEM)
        ]
        for d in descs:
            d.wait()

    plsc.subcore_barrier()
    pltpu.sync_copy(acc.at[pl.ds(sid * RPTD, RPTD)],
                    out_hbm.at[pl.ds(cid * ND_PAD + sid * RPTD, RPTD)])


def _prop_phase(g_hbm, out_hbm, p, acc, sidx, didx, rows, zbuf, sems, K):
    F = 64
    cid = lax.axis_index("c")
    sid = lax.axis_index("s")
    zd = [
        pltpu.async_copy(zbuf, acc.at[pl.ds(sid * RPT + z * ZR, ZR)], sems[z])
        for z in range(RPT // ZR)
    ]
    for d in zd:
        d.wait()
    plsc.subcore_barrier()

    @pl.loop(0, CHT // K)
    def _(r):
        c0 = r * K
        descs = [
            pltpu.async_copy(g_hbm.at[sidx.at[c0 + b]], rows.at[b], sems[b])
            for b in range(K)
        ]
        for b in range(K):
            descs[b].wait()
            pltpu.sync_copy(rows.at[b], acc.at[didx.at[c0 + b]], add=True)

    plsc.subcore_barrier()
    pltpu.sync_copy(
        acc.at[pl.ds(sid * RPT, RPT)],
        out_hbm.at[pl.ds((p * NC + cid) * N_PAD + sid * RPT, RPT)])


def _prop_scratch(K):
    return [
        pltpu.VMEM_SHARED((N_PAD, 64), jnp.float32),
        pltpu.VMEM((CHT, C), jnp.int32),
        pltpu.VMEM((CHT, C), jnp.int32),
        pltpu.VMEM((K, C, 64), jnp.float32),
        pltpu.VMEM((ZR, 64), jnp.float32),
    ] + [pltpu.SemaphoreType.DMA] * NSEM


_PROP_K = 5


@functools.partial(
    pl.kernel,
    out_type=jax.ShapeDtypeStruct((NC * N_PAD, 64), jnp.float32),
    mesh=_mesh(),
    scratch_types=_prop_scratch(_PROP_K),
    compiler_params=_SC_PARAMS,
)
def _prop64(g_hbm, src_hbm, dst_hbm, out_hbm, acc, sidx, didx, rows, zbuf,
            *sems):
    cid = lax.axis_index("c")
    sid = lax.axis_index("s")

    @pl.loop(0, ZR)
    def _(r):
        for q in range(4):
            zbuf[r, pl.ds(q * 16, 16)] = jnp.zeros((16,), jnp.float32)

    gwid = cid * NS + sid
    pltpu.sync_copy(src_hbm.at[pl.ds(gwid * CHT, CHT)], sidx)
    pltpu.sync_copy(dst_hbm.at[pl.ds(gwid * CHT, CHT)], didx)
    _prop_phase(g_hbm, out_hbm, 0, acc, sidx, didx, rows, zbuf, sems, _PROP_K)


@functools.partial(
    pl.kernel,
    out_type=jax.ShapeDtypeStruct((2 * NC * N_PAD, 64), jnp.float32),
    mesh=_mesh(),
    scratch_types=_prop_scratch(_PROP_K),
    compiler_params=_SC_PARAMS,
)
def _prop128(ga_hbm, gb_hbm, src_hbm, dst_hbm, out_hbm, acc, sidx, didx,
             rows, zbuf, *sems):
    cid = lax.axis_index("c")
    sid = lax.axis_index("s")

    @pl.loop(0, ZR)
    def _(r):
        for q in range(4):
            zbuf[r, pl.ds(q * 16, 16)] = jnp.zeros((16,), jnp.float32)

    gwid = cid * NS + sid
    pltpu.sync_copy(src_hbm.at[pl.ds(gwid * CHT, CHT)], sidx)
    pltpu.sync_copy(dst_hbm.at[pl.ds(gwid * CHT, CHT)], didx)
    _prop_phase(ga_hbm, out_hbm, 0, acc, sidx, didx, rows, zbuf, sems, _PROP_K)
    _prop_phase(gb_hbm, out_hbm, 1, acc, sidx, didx, rows, zbuf, sems, _PROP_K)


R = 1000


def _tc_first(x, W, c0, c1):
    Din, Dout = W.shape

    def body(x_ref, w_ref, c0_ref, c1_ref, ga_ref, gb_ref, dinv_ref):
        h = jnp.dot(x_ref[...], w_ref[...], preferred_element_type=jnp.float32)
        dinv = lax.rsqrt(c0_ref[...] + c1_ref[...] + 1.0)
        g = h * dinv
        ga_ref[...] = g[:, :Dout // 2]
        gb_ref[...] = g[:, Dout // 2:]
        dinv_ref[...] = dinv

    return pl.pallas_call(
        body,
        grid=(N // R,),
        in_specs=[
            pl.BlockSpec((R, Din), lambda i: (i, 0)),
            pl.BlockSpec((Din, Dout), lambda i: (0, 0)),
            pl.BlockSpec((R, 1), lambda i: (i, 0)),
            pl.BlockSpec((R, 1), lambda i: (i, 0)),
        ],
        out_specs=[
            pl.BlockSpec((R, Dout // 2), lambda i: (i, 0)),
            pl.BlockSpec((R, Dout // 2), lambda i: (i, 0)),
            pl.BlockSpec((R, 1), lambda i: (i, 0)),
        ],
        out_shape=[
            jax.ShapeDtypeStruct((N, Dout // 2), jnp.float32),
            jax.ShapeDtypeStruct((N, Dout // 2), jnp.float32),
            jax.ShapeDtypeStruct((N, 1), jnp.float32),
        ],
    )(x, W, c0, c1)


def _tc_mid2(aa0, aa1, ab0, ab1, ga, gb, dinv, ba, bb, Wa, Wb):
    Dh, Dout = Wa.shape

    def body(aa0_r, aa1_r, ab0_r, ab1_r, ga_r, gb_r, dinv_r, ba_r, bb_r,
             wa_r, wb_r, o_ref):
        dinv = dinv_r[...]
        sa = dinv * (aa0_r[...] + aa1_r[...] + ga_r[...]) + ba_r[...]
        sb = dinv * (ab0_r[...] + ab1_r[...] + gb_r[...]) + bb_r[...]
        acta = jnp.where(sa >= 0, sa, 0.2 * sa)
        actb = jnp.where(sb >= 0, sb, 0.2 * sb)
        h = (jnp.dot(acta, wa_r[...], preferred_element_type=jnp.float32)
             + jnp.dot(actb, wb_r[...], preferred_element_type=jnp.float32))
        o_ref[...] = h * dinv

    blk = lambda d: pl.BlockSpec((R, d), lambda i: (i, 0))
    cst = lambda s: pl.BlockSpec(s, lambda i: (0, 0))
    return pl.pallas_call(
        body,
        grid=(N // R,),
        in_specs=[blk(Dh), blk(Dh), blk(Dh), blk(Dh), blk(Dh), blk(Dh),
                  blk(1), cst((1, Dh)), cst((1, Dh)),
                  cst((Dh, Dout)), cst((Dh, Dout))],
        out_specs=pl.BlockSpec((R, Dout), lambda i: (i, 0)),
        out_shape=jax.ShapeDtypeStruct((N, Dout), jnp.float32),
    )(aa0, aa1, ab0, ab1, ga, gb, dinv, ba, bb, Wa, Wb)


def _tc_mid(a0, a1, g, dinv, b, W):
    Din, Dout = W.shape

    def body(a0_ref, a1_ref, g_ref, dinv_ref, b_ref, w_ref, o_ref):
        s = dinv_ref[...] * (a0_ref[...] + a1_ref[...] + g_ref[...]) + b_ref[...]
        act = jnp.where(s >= 0, s, 0.2 * s)
        h = jnp.dot(act, w_ref[...], preferred_element_type=jnp.float32)
        o_ref[...] = h * dinv_ref[...]

    return pl.pallas_call(
        body,
        grid=(N // R,),
        in_specs=[
            pl.BlockSpec((R, Din), lambda i: (i, 0)),
            pl.BlockSpec((R, Din), lambda i: (i, 0)),
            pl.BlockSpec((R, Din), lambda i: (i, 0)),
            pl.BlockSpec((R, 1), lambda i: (i, 0)),
            pl.BlockSpec((1, Din), lambda i: (0, 0)),
            pl.BlockSpec((Din, Dout), lambda i: (0, 0)),
        ],
        out_specs=pl.BlockSpec((R, Dout), lambda i: (i, 0)),
        out_shape=jax.ShapeDtypeStruct((N, Dout), jnp.float32),
    )(a0, a1, g, dinv, b, W)


def _tc_last(a0, a1, g, dinv, b):
    F = g.shape[1]

    def body(a0_ref, a1_ref, g_ref, dinv_ref, b_ref, o_ref):
        o_ref[...] = (dinv_ref[...] * (a0_ref[...] + a1_ref[...] + g_ref[...])
                      + b_ref[...])

    return pl.pallas_call(
        body,
        grid=(N // R,),
        in_specs=[
            pl.BlockSpec((R, F), lambda i: (i, 0)),
            pl.BlockSpec((R, F), lambda i: (i, 0)),
            pl.BlockSpec((R, F), lambda i: (i, 0)),
            pl.BlockSpec((R, 1), lambda i: (i, 0)),
            pl.BlockSpec((1, F), lambda i: (0, 0)),
        ],
        out_specs=pl.BlockSpec((R, F), lambda i: (i, 0)),
        out_shape=jax.ShapeDtypeStruct((N, F), jnp.float32),
    )(a0, a1, g, dinv, b)


def kernel(x, edge_index, W1, b1, W2, b2, W3, b3):
    ei = edge_index.astype(jnp.int32)
    npad = E_PAD - E
    src = jnp.concatenate([ei[0], jnp.zeros((npad,), jnp.int32)]).reshape(ECH, C)
    dst = jnp.concatenate([ei[1], jnp.full((npad,), N, jnp.int32)]).reshape(ECH, C)

    cnt = _deg(dst)
    c0 = cnt[:N].reshape(N, 1)
    c1 = cnt[ND_PAD:ND_PAD + N].reshape(N, 1)

    g1a, g1b, dinv = _tc_first(x, W1, c0, c1)
    acc = _prop128(g1a, g1b, src, dst)
    P = N_PAD
    g2 = _tc_mid2(acc[:N], acc[P:P + N], acc[2 * P:2 * P + N],
                  acc[3 * P:3 * P + N], g1a, g1b, dinv,
                  b1[:64].reshape(1, -1), b1[64:].reshape(1, -1),
                  W2[:64], W2[64:])
    acc = _prop64(g2, src, dst)
    g3 = _tc_mid(acc[:N], acc[P:P + N], g2, dinv, b2.reshape(1, -1), W3)
    acc = _prop64(g3, src, dst)
    return _tc_last(acc[:N], acc[P:P + N], g3, dinv, b3.reshape(1, -1))


# speedup vs baseline: 2.0034x; 2.0034x over previous
import functools

import jax
import jax.numpy as jnp
from jax import lax
from jax.experimental import pallas as pl
from jax.experimental.pallas import tpu as pltpu
from jax.experimental.pallas import tpu_sc as plsc

N = 10000
E = 160000
NC, NS = 2, 16
NW = NC * NS
C = 128
CHT = 40
E_PAD = NW * CHT * C
ECH = E_PAD // C
NSEM = 8

N_PAD = 10016
RPT = N_PAD // NS
ZR = 313
EPW = E // NW
PADW = CHT * C - EPW

ND_PAD = 10240
RPTD = ND_PAD // NS

_mesh = lambda: plsc.VectorSubcoreMesh(core_axis_name="c", subcore_axis_name="s")
_SC_PARAMS = pltpu.CompilerParams(use_tc_tiling_on_sc=False)


@functools.partial(
    pl.kernel,
    out_type=jax.ShapeDtypeStruct((NC * ND_PAD,), jnp.float32),
    mesh=_mesh(),
    scratch_types=[
        pltpu.VMEM_SHARED((ND_PAD,), jnp.float32),
        pltpu.VMEM((CHT, C), jnp.int32),
        pltpu.VMEM((C,), jnp.float32),
        pltpu.VMEM((RPTD,), jnp.float32),
    ] + [pltpu.SemaphoreType.DMA] * NSEM,
    compiler_params=_SC_PARAMS,
)
def _deg(dst_hbm, out_hbm, acc, didx, ones_v, zbuf, *sems):
    cid = lax.axis_index("c")
    sid = lax.axis_index("s")
    for i in range(C // 16):
        ones_v[pl.ds(i * 16, 16)] = jnp.full((16,), 1.0, jnp.float32)
    for i in range(RPTD // 16):
        zbuf[pl.ds(i * 16, 16)] = jnp.zeros((16,), jnp.float32)
    gwid = cid * NS + sid
    pltpu.sync_copy(dst_hbm.at[pl.ds(gwid * CHT, CHT)], didx)
    pltpu.sync_copy(zbuf, acc.at[pl.ds(sid * RPTD, RPTD)])
    plsc.subcore_barrier()

    @pl.loop(0, CHT // NSEM)
    def _(r):
        c0 = r * NSEM
        descs = [
            pltpu.async_copy(ones_v, acc.at[didx.at[c0 + b]], sems[b], add=True)
            for b in range(NSEM)
        ]
        for d in descs:
            d.wait()

    plsc.subcore_barrier()
    pltpu.sync_copy(acc.at[pl.ds(sid * RPTD, RPTD)],
                    out_hbm.at[pl.ds(cid * ND_PAD + sid * RPTD, RPTD)])


def _prop_phase(g_hbm, out_hbm, p, acc, sidx, didx, rows, zbuf, gsems, ssems,
                K):
    cid = lax.axis_index("c")
    sid = lax.axis_index("s")
    zd = [
        pltpu.async_copy(zbuf, acc.at[pl.ds(sid * RPT + z * ZR, ZR)], gsems[z])
        for z in range(RPT // ZR)
    ]
    for d in zd:
        d.wait()
    plsc.subcore_barrier()

    ROUNDS = CHT // K
    for b in range(K):
        pltpu.async_copy(g_hbm.at[sidx.at[b]], rows.at[b], gsems[b])

    @pl.loop(0, ROUNDS - 1)
    def _(r):
        c0 = r * K
        sds = []
        for b in range(K):
            pltpu.make_async_copy(g_hbm.at[sidx.at[c0 + b]], rows.at[b],
                                  gsems[b]).wait()
            sds.append(pltpu.async_copy(rows.at[b], acc.at[didx.at[c0 + b]],
                                        ssems[b], add=True))
        for b in range(K):
            sds[b].wait()
            pltpu.async_copy(g_hbm.at[sidx.at[c0 + K + b]], rows.at[b],
                             gsems[b])

    c0 = (ROUNDS - 1) * K
    sds = []
    for b in range(K):
        pltpu.make_async_copy(g_hbm.at[sidx.at[c0 + b]], rows.at[b],
                              gsems[b]).wait()
        sds.append(pltpu.async_copy(rows.at[b], acc.at[didx.at[c0 + b]],
                                    ssems[b], add=True))
    for d in sds:
        d.wait()

    plsc.subcore_barrier()
    pltpu.sync_copy(
        acc.at[pl.ds(sid * RPT, RPT)],
        out_hbm.at[pl.ds((p * NC + cid) * N_PAD + sid * RPT, RPT)])


def _prop_scratch(K):
    return [
        pltpu.VMEM_SHARED((N_PAD, 64), jnp.float32),
        pltpu.VMEM((CHT, C), jnp.int32),
        pltpu.VMEM((CHT, C), jnp.int32),
        pltpu.VMEM((K, C, 64), jnp.float32),
        pltpu.VMEM((ZR, 64), jnp.float32),
    ] + [pltpu.SemaphoreType.DMA] * (2 * K)


_PROP_K = 5


@functools.partial(
    pl.kernel,
    out_type=jax.ShapeDtypeStruct((NC * N_PAD, 64), jnp.float32),
    mesh=_mesh(),
    scratch_types=_prop_scratch(_PROP_K),
    compiler_params=_SC_PARAMS,
)
def _prop64(g_hbm, src_hbm, dst_hbm, out_hbm, acc, sidx, didx, rows, zbuf,
            *sems):
    cid = lax.axis_index("c")
    sid = lax.axis_index("s")

    @pl.loop(0, ZR)
    def _(r):
        for q in range(4):
            zbuf[r, pl.ds(q * 16, 16)] = jnp.zeros((16,), jnp.float32)

    gwid = cid * NS + sid
    pltpu.sync_copy(src_hbm.at[pl.ds(gwid * CHT, CHT)], sidx)
    pltpu.sync_copy(dst_hbm.at[pl.ds(gwid * CHT, CHT)], didx)
    _prop_phase(g_hbm, out_hbm, 0, acc, sidx, didx, rows, zbuf,
                sems[:_PROP_K], sems[_PROP_K:], _PROP_K)


@functools.partial(
    pl.kernel,
    out_type=jax.ShapeDtypeStruct((2 * NC * N_PAD, 64), jnp.float32),
    mesh=_mesh(),
    scratch_types=_prop_scratch(_PROP_K),
    compiler_params=_SC_PARAMS,
)
def _prop128(ga_hbm, gb_hbm, src_hbm, dst_hbm, out_hbm, acc, sidx, didx,
             rows, zbuf, *sems):
    cid = lax.axis_index("c")
    sid = lax.axis_index("s")

    @pl.loop(0, ZR)
    def _(r):
        for q in range(4):
            zbuf[r, pl.ds(q * 16, 16)] = jnp.zeros((16,), jnp.float32)

    gwid = cid * NS + sid
    pltpu.sync_copy(src_hbm.at[pl.ds(gwid * CHT, CHT)], sidx)
    pltpu.sync_copy(dst_hbm.at[pl.ds(gwid * CHT, CHT)], didx)
    _prop_phase(ga_hbm, out_hbm, 0, acc, sidx, didx, rows, zbuf,
                sems[:_PROP_K], sems[_PROP_K:], _PROP_K)
    _prop_phase(gb_hbm, out_hbm, 1, acc, sidx, didx, rows, zbuf,
                sems[:_PROP_K], sems[_PROP_K:], _PROP_K)


R = 1000


def _tc_first(x, W, c0, c1):
    Din, Dout = W.shape

    def body(x_ref, w_ref, c0_ref, c1_ref, ga_ref, gb_ref, dinv_ref):
        h = jnp.dot(x_ref[...], w_ref[...], preferred_element_type=jnp.float32)
        dinv = lax.rsqrt(c0_ref[...] + c1_ref[...] + 1.0)
        g = h * dinv
        ga_ref[...] = g[:, :Dout // 2]
        gb_ref[...] = g[:, Dout // 2:]
        dinv_ref[...] = dinv

    return pl.pallas_call(
        body,
        grid=(N // R,),
        in_specs=[
            pl.BlockSpec((R, Din), lambda i: (i, 0)),
            pl.BlockSpec((Din, Dout), lambda i: (0, 0)),
            pl.BlockSpec((R, 1), lambda i: (i, 0)),
            pl.BlockSpec((R, 1), lambda i: (i, 0)),
        ],
        out_specs=[
            pl.BlockSpec((R, Dout // 2), lambda i: (i, 0)),
            pl.BlockSpec((R, Dout // 2), lambda i: (i, 0)),
            pl.BlockSpec((R, 1), lambda i: (i, 0)),
        ],
        out_shape=[
            jax.ShapeDtypeStruct((N, Dout // 2), jnp.float32),
            jax.ShapeDtypeStruct((N, Dout // 2), jnp.float32),
            jax.ShapeDtypeStruct((N, 1), jnp.float32),
        ],
    )(x, W, c0, c1)


def _tc_mid2(aa0, aa1, ab0, ab1, ga, gb, dinv, ba, bb, Wa, Wb):
    Dh, Dout = Wa.shape

    def body(aa0_r, aa1_r, ab0_r, ab1_r, ga_r, gb_r, dinv_r, ba_r, bb_r,
             wa_r, wb_r, o_ref):
        dinv = dinv_r[...]
        sa = dinv * (aa0_r[...] + aa1_r[...] + ga_r[...]) + ba_r[...]
        sb = dinv * (ab0_r[...] + ab1_r[...] + gb_r[...]) + bb_r[...]
        acta = jnp.where(sa >= 0, sa, 0.2 * sa)
        actb = jnp.where(sb >= 0, sb, 0.2 * sb)
        h = (jnp.dot(acta, wa_r[...], preferred_element_type=jnp.float32)
             + jnp.dot(actb, wb_r[...], preferred_element_type=jnp.float32))
        o_ref[...] = h * dinv

    blk = lambda d: pl.BlockSpec((R, d), lambda i: (i, 0))
    cst = lambda s: pl.BlockSpec(s, lambda i: (0, 0))
    return pl.pallas_call(
        body,
        grid=(N // R,),
        in_specs=[blk(Dh), blk(Dh), blk(Dh), blk(Dh), blk(Dh), blk(Dh),
                  blk(1), cst((1, Dh)), cst((1, Dh)),
                  cst((Dh, Dout)), cst((Dh, Dout))],
        out_specs=pl.BlockSpec((R, Dout), lambda i: (i, 0)),
        out_shape=jax.ShapeDtypeStruct((N, Dout), jnp.float32),
    )(aa0, aa1, ab0, ab1, ga, gb, dinv, ba, bb, Wa, Wb)


def _tc_mid(a0, a1, g, dinv, b, W):
    Din, Dout = W.shape

    def body(a0_ref, a1_ref, g_ref, dinv_ref, b_ref, w_ref, o_ref):
        s = dinv_ref[...] * (a0_ref[...] + a1_ref[...] + g_ref[...]) + b_ref[...]
        act = jnp.where(s >= 0, s, 0.2 * s)
        h = jnp.dot(act, w_ref[...], preferred_element_type=jnp.float32)
        o_ref[...] = h * dinv_ref[...]

    return pl.pallas_call(
        body,
        grid=(N // R,),
        in_specs=[
            pl.BlockSpec((R, Din), lambda i: (i, 0)),
            pl.BlockSpec((R, Din), lambda i: (i, 0)),
            pl.BlockSpec((R, Din), lambda i: (i, 0)),
            pl.BlockSpec((R, 1), lambda i: (i, 0)),
            pl.BlockSpec((1, Din), lambda i: (0, 0)),
            pl.BlockSpec((Din, Dout), lambda i: (0, 0)),
        ],
        out_specs=pl.BlockSpec((R, Dout), lambda i: (i, 0)),
        out_shape=jax.ShapeDtypeStruct((N, Dout), jnp.float32),
    )(a0, a1, g, dinv, b, W)


def _tc_last(a0, a1, g, dinv, b):
    F = g.shape[1]

    def body(a0_ref, a1_ref, g_ref, dinv_ref, b_ref, o_ref):
        o_ref[...] = (dinv_ref[...] * (a0_ref[...] + a1_ref[...] + g_ref[...])
                      + b_ref[...])

    return pl.pallas_call(
        body,
        grid=(N // R,),
        in_specs=[
            pl.BlockSpec((R, F), lambda i: (i, 0)),
            pl.BlockSpec((R, F), lambda i: (i, 0)),
            pl.BlockSpec((R, F), lambda i: (i, 0)),
            pl.BlockSpec((R, 1), lambda i: (i, 0)),
            pl.BlockSpec((1, F), lambda i: (0, 0)),
        ],
        out_specs=pl.BlockSpec((R, F), lambda i: (i, 0)),
        out_shape=jax.ShapeDtypeStruct((N, F), jnp.float32),
    )(a0, a1, g, dinv, b)


def kernel(x, edge_index, W1, b1, W2, b2, W3, b3):
    ei = edge_index.astype(jnp.int32)
    spread = jnp.arange(PADW, dtype=jnp.int32) % 16
    pad_s = jnp.broadcast_to(spread, (NW, PADW))
    pad_d = jnp.broadcast_to(N + spread, (NW, PADW))
    src = jnp.concatenate([ei[0].reshape(NW, EPW), pad_s], axis=1).reshape(ECH, C)
    dst = jnp.concatenate([ei[1].reshape(NW, EPW), pad_d], axis=1).reshape(ECH, C)

    cnt = _deg(dst)
    c0 = cnt[:N].reshape(N, 1)
    c1 = cnt[ND_PAD:ND_PAD + N].reshape(N, 1)

    g1a, g1b, dinv = _tc_first(x, W1, c0, c1)
    acc = _prop128(g1a, g1b, src, dst)
    P = N_PAD
    g2 = _tc_mid2(acc[:N], acc[P:P + N], acc[2 * P:2 * P + N],
                  acc[3 * P:3 * P + N], g1a, g1b, dinv,
                  b1[:64].reshape(1, -1), b1[64:].reshape(1, -1),
                  W2[:64], W2[64:])
    acc = _prop64(g2, src, dst)
    g3 = _tc_mid(acc[:N], acc[P:P + N], g2, dinv, b2.reshape(1, -1), W3)
    acc = _prop64(g3, src, dst)
    return _tc_last(acc[:N], acc[P:P + N], g3, dinv, b3.reshape(1, -1))


# speedup vs baseline: 2.1966x; 1.0965x over previous
import functools

import jax
import jax.numpy as jnp
from jax import lax
from jax.experimental import pallas as pl
from jax.experimental.pallas import tpu as pltpu
from jax.experimental.pallas import tpu_sc as plsc

N = 10000
E = 160000
NC, NS = 2, 16
NW = NC * NS
C = 128
CHT = 40
E_PAD = NW * CHT * C
ECH = E_PAD // C
NSEM = 8

N_PAD = 10240
RPT = N_PAD // NS
ZR = 320
EPW = E // NW
PADW = CHT * C - EPW

ND_PAD = 10240
RPTD = ND_PAD // NS

_mesh = lambda: plsc.VectorSubcoreMesh(core_axis_name="c", subcore_axis_name="s")
_SC_LINEAR = pltpu.CompilerParams(use_tc_tiling_on_sc=False)


@functools.partial(
    pl.kernel,
    out_type=jax.ShapeDtypeStruct((NC * ND_PAD,), jnp.float32),
    mesh=_mesh(),
    scratch_types=[
        pltpu.VMEM_SHARED((ND_PAD,), jnp.float32),
        pltpu.VMEM((CHT, C), jnp.int32),
        pltpu.VMEM((C,), jnp.float32),
        pltpu.VMEM((RPTD,), jnp.float32),
    ] + [pltpu.SemaphoreType.DMA] * NSEM,
)
def _deg(dst_hbm, out_hbm, acc, didx, ones_v, zbuf, *sems):
    cid = lax.axis_index("c")
    sid = lax.axis_index("s")
    for i in range(C // 16):
        ones_v[pl.ds(i * 16, 16)] = jnp.full((16,), 1.0, jnp.float32)
    for i in range(RPTD // 16):
        zbuf[pl.ds(i * 16, 16)] = jnp.zeros((16,), jnp.float32)
    gwid = cid * NS + sid
    pltpu.sync_copy(dst_hbm.at[pl.ds(gwid * CHT, CHT)], didx)
    pltpu.sync_copy(zbuf, acc.at[pl.ds(sid * RPTD, RPTD)])
    plsc.subcore_barrier()

    @pl.loop(0, CHT // NSEM)
    def _(r):
        c0 = r * NSEM
        descs = [
            pltpu.async_copy(ones_v, acc.at[didx.at[c0 + b]], sems[b], add=True)
            for b in range(NSEM)
        ]
        for d in descs:
            d.wait()

    plsc.subcore_barrier()
    pltpu.sync_copy(acc.at[pl.ds(sid * RPTD, RPTD)],
                    out_hbm.at[pl.ds(cid * ND_PAD + sid * RPTD, RPTD)])


def _prop_phase(g_hbm, out_hbm, p, acc, sidx, didx, rows, zbuf, gsems, ssems,
                K):
    cid = lax.axis_index("c")
    sid = lax.axis_index("s")
    zd = [
        pltpu.async_copy(zbuf, acc.at[pl.ds(sid * RPT + z * ZR, ZR)], gsems[z])
        for z in range(RPT // ZR)
    ]
    for d in zd:
        d.wait()
    plsc.subcore_barrier()

    ROUNDS = CHT // K
    for b in range(K):
        pltpu.async_copy(g_hbm.at[sidx.at[b]], rows.at[b], gsems[b])

    @pl.loop(0, ROUNDS - 1)
    def _(r):
        c0 = r * K
        sds = []
        for b in range(K):
            pltpu.make_async_copy(g_hbm.at[sidx.at[c0 + b]], rows.at[b],
                                  gsems[b]).wait()
            sds.append(pltpu.async_copy(rows.at[b], acc.at[didx.at[c0 + b]],
                                        ssems[b], add=True))
        for b in range(K):
            sds[b].wait()
            pltpu.async_copy(g_hbm.at[sidx.at[c0 + K + b]], rows.at[b],
                             gsems[b])

    c0 = (ROUNDS - 1) * K
    sds = []
    for b in range(K):
        pltpu.make_async_copy(g_hbm.at[sidx.at[c0 + b]], rows.at[b],
                              gsems[b]).wait()
        sds.append(pltpu.async_copy(rows.at[b], acc.at[didx.at[c0 + b]],
                                    ssems[b], add=True))
    for d in sds:
        d.wait()

    plsc.subcore_barrier()
    pltpu.sync_copy(
        acc.at[pl.ds(sid * RPT, RPT)],
        out_hbm.at[pl.ds((p * NC + cid) * N_PAD + sid * RPT, RPT)])


def _prop_scratch(K):
    return [
        pltpu.VMEM_SHARED((N_PAD, 64), jnp.float32),
        pltpu.VMEM((CHT, C), jnp.int32),
        pltpu.VMEM((CHT, C), jnp.int32),
        pltpu.VMEM((K, C, 64), jnp.float32),
        pltpu.VMEM((ZR, 64), jnp.float32),
    ] + [pltpu.SemaphoreType.DMA] * (2 * K)


_PROP_K = 5


@functools.partial(
    pl.kernel,
    out_type=jax.ShapeDtypeStruct((NC * N_PAD, 64), jnp.float32),
    mesh=_mesh(),
    scratch_types=_prop_scratch(_PROP_K),
    compiler_params=_SC_LINEAR,
)
def _prop64(g_hbm, src_hbm, dst_hbm, out_hbm, acc, sidx, didx, rows, zbuf,
            *sems):
    cid = lax.axis_index("c")
    sid = lax.axis_index("s")

    @pl.loop(0, ZR)
    def _(r):
        for q in range(4):
            zbuf[r, pl.ds(q * 16, 16)] = jnp.zeros((16,), jnp.float32)

    gwid = cid * NS + sid
    pltpu.sync_copy(src_hbm.at[pl.ds(gwid * CHT, CHT)], sidx)
    pltpu.sync_copy(dst_hbm.at[pl.ds(gwid * CHT, CHT)], didx)
    _prop_phase(g_hbm, out_hbm, 0, acc, sidx, didx, rows, zbuf,
                sems[:_PROP_K], sems[_PROP_K:], _PROP_K)


_K128 = 2


@functools.partial(
    pl.kernel,
    out_type=jax.ShapeDtypeStruct((NC * N_PAD, 128), jnp.float32),
    mesh=_mesh(),
    scratch_types=[
        pltpu.VMEM_SHARED((N_PAD, 128), jnp.float32),
        pltpu.VMEM((CHT, C), jnp.int32),
        pltpu.VMEM((CHT, C), jnp.int32),
        pltpu.VMEM((_K128, C, 128), jnp.float32),
    ] + [pltpu.SemaphoreType.DMA] * (2 * _K128 + 1),
)
def _prop128(g_hbm, src_hbm, dst_hbm, out_hbm, acc, sidx, didx, rows, *sems):
    K = _K128
    gsems, ssems, zsem = sems[:K], sems[K:2 * K], sems[2 * K]
    cid = lax.axis_index("c")
    sid = lax.axis_index("s")

    @pl.loop(0, C)
    def _(r):
        for q in range(8):
            rows[0, r, pl.ds(q * 16, 16)] = jnp.zeros((16,), jnp.float32)

    gwid = cid * NS + sid
    pltpu.sync_copy(src_hbm.at[pl.ds(gwid * CHT, CHT)], sidx)
    pltpu.sync_copy(dst_hbm.at[pl.ds(gwid * CHT, CHT)], didx)
    zd = [
        pltpu.async_copy(rows.at[0], acc.at[pl.ds(sid * RPT + z * C, C)], zsem)
        for z in range(RPT // C)
    ]
    for d in zd:
        d.wait()
    plsc.subcore_barrier()

    ROUNDS = CHT // K
    for b in range(K):
        pltpu.async_copy(g_hbm.at[sidx.at[b]], rows.at[b], gsems[b])

    @pl.loop(0, ROUNDS - 1)
    def _(r):
        c0 = r * K
        sds = []
        for b in range(K):
            pltpu.make_async_copy(g_hbm.at[sidx.at[c0 + b]], rows.at[b],
                                  gsems[b]).wait()
            sds.append(pltpu.async_copy(rows.at[b], acc.at[didx.at[c0 + b]],
                                        ssems[b], add=True))
        for b in range(K):
            sds[b].wait()
            pltpu.async_copy(g_hbm.at[sidx.at[c0 + K + b]], rows.at[b],
                             gsems[b])

    c0 = (ROUNDS - 1) * K
    sds = []
    for b in range(K):
        pltpu.make_async_copy(g_hbm.at[sidx.at[c0 + b]], rows.at[b],
                              gsems[b]).wait()
        sds.append(pltpu.async_copy(rows.at[b], acc.at[didx.at[c0 + b]],
                                    ssems[b], add=True))
    for d in sds:
        d.wait()

    plsc.subcore_barrier()
    pltpu.sync_copy(acc.at[pl.ds(sid * RPT, RPT)],
                    out_hbm.at[pl.ds(cid * N_PAD + sid * RPT, RPT)])


R = 1000


def _tc_first(x, W, c0, c1):
    Din, Dout = W.shape

    def body(x_ref, w_ref, c0_ref, c1_ref, g_ref, dinv_ref):
        h = jnp.dot(x_ref[...], w_ref[...], preferred_element_type=jnp.float32)
        dinv = lax.rsqrt(c0_ref[...] + c1_ref[...] + 1.0)
        g_ref[...] = h * dinv
        dinv_ref[...] = dinv

    return pl.pallas_call(
        body,
        grid=(N // R,),
        in_specs=[
            pl.BlockSpec((R, Din), lambda i: (i, 0)),
            pl.BlockSpec((Din, Dout), lambda i: (0, 0)),
            pl.BlockSpec((R, 1), lambda i: (i, 0)),
            pl.BlockSpec((R, 1), lambda i: (i, 0)),
        ],
        out_specs=[
            pl.BlockSpec((R, Dout), lambda i: (i, 0)),
            pl.BlockSpec((R, 1), lambda i: (i, 0)),
        ],
        out_shape=[
            jax.ShapeDtypeStruct((N, Dout), jnp.float32),
            jax.ShapeDtypeStruct((N, 1), jnp.float32),
        ],
    )(x, W, c0, c1)


def _tc_mid(a0, a1, g, dinv, b, W):
    Din, Dout = W.shape

    def body(a0_ref, a1_ref, g_ref, dinv_ref, b_ref, w_ref, o_ref):
        s = dinv_ref[...] * (a0_ref[...] + a1_ref[...] + g_ref[...]) + b_ref[...]
        act = jnp.where(s >= 0, s, 0.2 * s)
        h = jnp.dot(act, w_ref[...], preferred_element_type=jnp.float32)
        o_ref[...] = h * dinv_ref[...]

    return pl.pallas_call(
        body,
        grid=(N // R,),
        in_specs=[
            pl.BlockSpec((R, Din), lambda i: (i, 0)),
            pl.BlockSpec((R, Din), lambda i: (i, 0)),
            pl.BlockSpec((R, Din), lambda i: (i, 0)),
            pl.BlockSpec((R, 1), lambda i: (i, 0)),
            pl.BlockSpec((1, Din), lambda i: (0, 0)),
            pl.BlockSpec((Din, Dout), lambda i: (0, 0)),
        ],
        out_specs=pl.BlockSpec((R, Dout), lambda i: (i, 0)),
        out_shape=jax.ShapeDtypeStruct((N, Dout), jnp.float32),
    )(a0, a1, g, dinv, b, W)


def _tc_last(a0, a1, g, dinv, b):
    F = g.shape[1]

    def body(a0_ref, a1_ref, g_ref, dinv_ref, b_ref, o_ref):
        o_ref[...] = (dinv_ref[...] * (a0_ref[...] + a1_ref[...] + g_ref[...])
                      + b_ref[...])

    return pl.pallas_call(
        body,
        grid=(N // R,),
        in_specs=[
            pl.BlockSpec((R, F), lambda i: (i, 0)),
            pl.BlockSpec((R, F), lambda i: (i, 0)),
            pl.BlockSpec((R, F), lambda i: (i, 0)),
            pl.BlockSpec((R, 1), lambda i: (i, 0)),
            pl.BlockSpec((1, F), lambda i: (0, 0)),
        ],
        out_specs=pl.BlockSpec((R, F), lambda i: (i, 0)),
        out_shape=jax.ShapeDtypeStruct((N, F), jnp.float32),
    )(a0, a1, g, dinv, b)


def kernel(x, edge_index, W1, b1, W2, b2, W3, b3):
    ei = edge_index.astype(jnp.int32)
    spread = jnp.arange(PADW, dtype=jnp.int32) % 16
    pad_s = jnp.broadcast_to(spread, (NW, PADW))
    pad_d = jnp.broadcast_to(N + spread, (NW, PADW))
    src = jnp.concatenate([ei[0].reshape(NW, EPW), pad_s], axis=1).reshape(ECH, C)
    dst = jnp.concatenate([ei[1].reshape(NW, EPW), pad_d], axis=1).reshape(ECH, C)

    cnt = _deg(dst)
    c0 = cnt[:N].reshape(N, 1)
    c1 = cnt[ND_PAD:ND_PAD + N].reshape(N, 1)

    g1, dinv = _tc_first(x, W1, c0, c1)
    acc = _prop128(g1, src, dst)
    P = N_PAD
    g2 = _tc_mid(acc[:N], acc[P:P + N], g1, dinv, b1.reshape(1, -1), W2)
    acc = _prop64(g2, src, dst)
    g3 = _tc_mid(acc[:N], acc[P:P + N], g2, dinv, b2.reshape(1, -1), W3)
    acc = _prop64(g3, src, dst)
    return _tc_last(acc[:N], acc[P:P + N], g3, dinv, b3.reshape(1, -1))


# speedup vs baseline: 2.3858x; 1.0861x over previous
import functools

import jax
import jax.numpy as jnp
from jax import lax
from jax.experimental import pallas as pl
from jax.experimental.pallas import tpu as pltpu
from jax.experimental.pallas import tpu_sc as plsc

N = 10000
E = 160000
NC, NS = 2, 16
NW = NC * NS
C = 128
CHT = 40
EPW = E // NW
NFULL = EPW // C
TAIL = EPW - NFULL * C

N_PAD = 10240
RPT = N_PAD // NS
ZR = 320

_mesh = lambda: plsc.VectorSubcoreMesh(core_axis_name="c", subcore_axis_name="s")
_SC_LINEAR = pltpu.CompilerParams(use_tc_tiling_on_sc=False)


def _stage_idx(src_hbm, dst_hbm, sidx, didx, base, sem):
    iota = lax.iota(jnp.int32, 16)
    for q in range(C // 16):
        sidx[NFULL, pl.ds(q * 16, 16)] = iota
        didx[NFULL, pl.ds(q * 16, 16)] = iota + N
    descs = []
    for j in range(NFULL):
        descs.append(pltpu.async_copy(
            src_hbm.at[pl.ds(base + j * C, C)], sidx.at[j], sem))
        descs.append(pltpu.async_copy(
            dst_hbm.at[pl.ds(base + j * C, C)], didx.at[j], sem))
    descs.append(pltpu.async_copy(
        src_hbm.at[pl.ds(base + NFULL * C, TAIL)],
        sidx.at[NFULL, pl.ds(0, TAIL)], sem))
    descs.append(pltpu.async_copy(
        dst_hbm.at[pl.ds(base + NFULL * C, TAIL)],
        didx.at[NFULL, pl.ds(0, TAIL)], sem))
    for d in descs:
        d.wait()


@functools.partial(
    pl.kernel,
    out_type=jax.ShapeDtypeStruct((NC * N_PAD,), jnp.float32),
    mesh=_mesh(),
    scratch_types=[
        pltpu.VMEM_SHARED((N_PAD,), jnp.float32),
        pltpu.VMEM((CHT, C), jnp.int32),
        pltpu.VMEM((CHT, C), jnp.int32),
        pltpu.VMEM((C,), jnp.float32),
        pltpu.VMEM((RPT,), jnp.float32),
    ] + [pltpu.SemaphoreType.DMA] * 8,
)
def _deg(src_hbm, dst_hbm, out_hbm, acc, didx, sidx, ones_v, zbuf, *sems):
    cid = lax.axis_index("c")
    sid = lax.axis_index("s")
    for i in range(C // 16):
        ones_v[pl.ds(i * 16, 16)] = jnp.full((16,), 1.0, jnp.float32)
    for i in range(RPT // 16):
        zbuf[pl.ds(i * 16, 16)] = jnp.zeros((16,), jnp.float32)
    base = (cid * NS + sid) * EPW
    _stage_idx(src_hbm, dst_hbm, sidx, didx, base, sems[0])
    pltpu.sync_copy(zbuf, acc.at[pl.ds(sid * RPT, RPT)])
    plsc.subcore_barrier()

    @pl.loop(0, CHT // 8)
    def _(r):
        c0 = r * 8
        descs = [
            pltpu.async_copy(ones_v, acc.at[didx.at[c0 + b]], sems[b], add=True)
            for b in range(8)
        ]
        for d in descs:
            d.wait()

    plsc.subcore_barrier()
    pltpu.sync_copy(acc.at[pl.ds(sid * RPT, RPT)],
                    out_hbm.at[pl.ds(cid * N_PAD + sid * RPT, RPT)])


def _prop_loop(g_hbm, acc, sidx, didx, rows, gsems, ssems, K):
    ROUNDS = CHT // K
    for b in range(K):
        pltpu.async_copy(g_hbm.at[sidx.at[b]], rows.at[b], gsems[b])

    @pl.loop(0, ROUNDS - 1)
    def _(r):
        c0 = r * K
        sds = []
        for b in range(K):
            pltpu.make_async_copy(g_hbm.at[sidx.at[c0 + b]], rows.at[b],
                                  gsems[b]).wait()
            sds.append(pltpu.async_copy(rows.at[b], acc.at[didx.at[c0 + b]],
                                        ssems[b], add=True))
        for b in range(K):
            sds[b].wait()
            pltpu.async_copy(g_hbm.at[sidx.at[c0 + K + b]], rows.at[b],
                             gsems[b])

    c0 = (ROUNDS - 1) * K
    sds = []
    for b in range(K):
        pltpu.make_async_copy(g_hbm.at[sidx.at[c0 + b]], rows.at[b],
                              gsems[b]).wait()
        sds.append(pltpu.async_copy(rows.at[b], acc.at[didx.at[c0 + b]],
                                    ssems[b], add=True))
    for d in sds:
        d.wait()


_K64 = 5
_K128 = 2


@functools.partial(
    pl.kernel,
    out_type=jax.ShapeDtypeStruct((NC * N_PAD, 64), jnp.float32),
    mesh=_mesh(),
    scratch_types=[
        pltpu.VMEM_SHARED((N_PAD, 64), jnp.float32),
        pltpu.VMEM((CHT, C), jnp.int32),
        pltpu.VMEM((CHT, C), jnp.int32),
        pltpu.VMEM((_K64, C, 64), jnp.float32),
        pltpu.VMEM((ZR, 64), jnp.float32),
    ] + [pltpu.SemaphoreType.DMA] * (2 * _K64),
    compiler_params=_SC_LINEAR,
)
def _prop64(g_hbm, src_hbm, dst_hbm, out_hbm, acc, sidx, didx, rows, zbuf,
            *sems):
    K = _K64
    cid = lax.axis_index("c")
    sid = lax.axis_index("s")

    @pl.loop(0, ZR)
    def _(r):
        for q in range(4):
            zbuf[r, pl.ds(q * 16, 16)] = jnp.zeros((16,), jnp.float32)

    base = (cid * NS + sid) * EPW
    _stage_idx(src_hbm, dst_hbm, sidx, didx, base, sems[0])
    zd = [
        pltpu.async_copy(zbuf, acc.at[pl.ds(sid * RPT + z * ZR, ZR)], sems[z])
        for z in range(RPT // ZR)
    ]
    for d in zd:
        d.wait()
    plsc.subcore_barrier()

    _prop_loop(g_hbm, acc, sidx, didx, rows, sems[:K], sems[K:], K)

    plsc.subcore_barrier()
    pltpu.sync_copy(acc.at[pl.ds(sid * RPT, RPT)],
                    out_hbm.at[pl.ds(cid * N_PAD + sid * RPT, RPT)])


@functools.partial(
    pl.kernel,
    out_type=jax.ShapeDtypeStruct((NC * N_PAD, 128), jnp.float32),
    mesh=_mesh(),
    scratch_types=[
        pltpu.VMEM_SHARED((N_PAD, 128), jnp.float32),
        pltpu.VMEM((CHT, C), jnp.int32),
        pltpu.VMEM((CHT, C), jnp.int32),
        pltpu.VMEM((_K128, C, 128), jnp.float32),
    ] + [pltpu.SemaphoreType.DMA] * (2 * _K128 + 1),
)
def _prop128(g_hbm, src_hbm, dst_hbm, out_hbm, acc, sidx, didx, rows, *sems):
    K = _K128
    gsems, ssems, zsem = sems[:K], sems[K:2 * K], sems[2 * K]
    cid = lax.axis_index("c")
    sid = lax.axis_index("s")

    @pl.loop(0, C)
    def _(r):
        for q in range(8):
            rows[0, r, pl.ds(q * 16, 16)] = jnp.zeros((16,), jnp.float32)

    base = (cid * NS + sid) * EPW
    _stage_idx(src_hbm, dst_hbm, sidx, didx, base, zsem)
    zd = [
        pltpu.async_copy(rows.at[0], acc.at[pl.ds(sid * RPT + z * C, C)], zsem)
        for z in range(RPT // C)
    ]
    for d in zd:
        d.wait()
    plsc.subcore_barrier()

    _prop_loop(g_hbm, acc, sidx, didx, rows, gsems, ssems, K)

    plsc.subcore_barrier()
    pltpu.sync_copy(acc.at[pl.ds(sid * RPT, RPT)],
                    out_hbm.at[pl.ds(cid * N_PAD + sid * RPT, RPT)])


R = 1024
GRID = N_PAD // R


def _tc_first(x, W, c0, c1):
    Din, Dout = W.shape

    def body(x_ref, w_ref, c0_ref, c1_ref, g_ref, dinv_ref):
        h = jnp.dot(x_ref[...], w_ref[...], preferred_element_type=jnp.float32)
        dinv = lax.rsqrt(c0_ref[...] + c1_ref[...] + 1.0)
        dinv_ref[...] = dinv
        g_ref[...] = h * dinv

    return pl.pallas_call(
        body,
        grid=(GRID,),
        in_specs=[
            pl.BlockSpec((R, Din), lambda i: (i, 0)),
            pl.BlockSpec((Din, Dout), lambda i: (0, 0)),
            pl.BlockSpec((R, 1), lambda i: (i, 0)),
            pl.BlockSpec((R, 1), lambda i: (i, 0)),
        ],
        out_specs=[
            pl.BlockSpec((R, Dout), lambda i: (i, 0)),
            pl.BlockSpec((R, 1), lambda i: (i, 0)),
        ],
        out_shape=[
            jax.ShapeDtypeStruct((N, Dout), jnp.float32),
            jax.ShapeDtypeStruct((N_PAD, 1), jnp.float32),
        ],
    )(x, W, c0, c1)


def _tc_mid(acc, g, dinv, b, W):
    Din, Dout = W.shape

    def body(a0_ref, a1_ref, g_ref, dv_ref, b_ref, w_ref, o_ref):
        dcol = dv_ref[...]
        s = dcol * (a0_ref[...] + a1_ref[...] + g_ref[...]) + b_ref[...]
        act = jnp.where(s >= 0, s, 0.2 * s)
        h = jnp.dot(act, w_ref[...], preferred_element_type=jnp.float32)
        o_ref[...] = h * dcol

    return pl.pallas_call(
        body,
        grid=(GRID,),
        in_specs=[
            pl.BlockSpec((R, Din), lambda i: (i, 0)),
            pl.BlockSpec((R, Din), lambda i: (i + GRID, 0)),
            pl.BlockSpec((R, Din), lambda i: (i, 0)),
            pl.BlockSpec((R, 1), lambda i: (i, 0)),
            pl.BlockSpec((1, Din), lambda i: (0, 0)),
            pl.BlockSpec((Din, Dout), lambda i: (0, 0)),
        ],
        out_specs=pl.BlockSpec((R, Dout), lambda i: (i, 0)),
        out_shape=jax.ShapeDtypeStruct((N, Dout), jnp.float32),
    )(acc, acc, g, dinv, b, W)


def _tc_last(acc, g, dinv, b):
    F = g.shape[1]

    def body(a0_ref, a1_ref, g_ref, dv_ref, b_ref, o_ref):
        o_ref[...] = (dv_ref[...] * (a0_ref[...] + a1_ref[...] + g_ref[...])
                      + b_ref[...])

    return pl.pallas_call(
        body,
        grid=(GRID,),
        in_specs=[
            pl.BlockSpec((R, F), lambda i: (i, 0)),
            pl.BlockSpec((R, F), lambda i: (i + GRID, 0)),
            pl.BlockSpec((R, F), lambda i: (i, 0)),
            pl.BlockSpec((R, 1), lambda i: (i, 0)),
            pl.BlockSpec((1, F), lambda i: (0, 0)),
        ],
        out_specs=pl.BlockSpec((R, F), lambda i: (i, 0)),
        out_shape=jax.ShapeDtypeStruct((N, F), jnp.float32),
    )(acc, acc, g, dinv, b)


def kernel(x, edge_index, W1, b1, W2, b2, W3, b3):
    ei = edge_index.astype(jnp.int32)
    src, dst = ei[0], ei[1]

    cnt = _deg(src, dst)
    c0 = cnt[:N_PAD].reshape(N_PAD, 1)
    c1 = cnt[N_PAD:].reshape(N_PAD, 1)

    g1, dinv = _tc_first(x, W1, c0, c1)
    acc = _prop128(g1, src, dst)
    g2 = _tc_mid(acc, g1, dinv, b1.reshape(1, -1), W2)
    acc = _prop64(g2, src, dst)
    g3 = _tc_mid(acc, g2, dinv, b2.reshape(1, -1), W3)
    acc = _prop64(g3, src, dst)
    return _tc_last(acc, g3, dinv, b3.reshape(1, -1))


# speedup vs baseline: 2.4235x; 1.0158x over previous
import functools

import jax
import jax.numpy as jnp
from jax import lax
from jax.experimental import pallas as pl
from jax.experimental.pallas import tpu as pltpu
from jax.experimental.pallas import tpu_sc as plsc

N = 10000
E = 160000
NC, NS = 2, 16
NW = NC * NS
C = 128
CHT = 40
EPW = E // NW
NFULL = EPW // C
TAIL = EPW - NFULL * C

N_PAD = 10240
RPT = N_PAD // NS
ZR = 160

_mesh = lambda: plsc.VectorSubcoreMesh(core_axis_name="c", subcore_axis_name="s")
_SC_LINEAR = pltpu.CompilerParams(use_tc_tiling_on_sc=False)


def _stage_idx(src_hbm, dst_hbm, sidx, didx, base, sem):
    iota = lax.iota(jnp.int32, 16)
    for q in range(C // 16):
        sidx[NFULL, pl.ds(q * 16, 16)] = iota
        didx[NFULL, pl.ds(q * 16, 16)] = iota + N
    descs = []
    for j in range(NFULL):
        descs.append(pltpu.async_copy(
            src_hbm.at[pl.ds(base + j * C, C)], sidx.at[j], sem))
        descs.append(pltpu.async_copy(
            dst_hbm.at[pl.ds(base + j * C, C)], didx.at[j], sem))
    descs.append(pltpu.async_copy(
        src_hbm.at[pl.ds(base + NFULL * C, TAIL)],
        sidx.at[NFULL, pl.ds(0, TAIL)], sem))
    descs.append(pltpu.async_copy(
        dst_hbm.at[pl.ds(base + NFULL * C, TAIL)],
        didx.at[NFULL, pl.ds(0, TAIL)], sem))
    for d in descs:
        d.wait()


@functools.partial(
    pl.kernel,
    out_type=jax.ShapeDtypeStruct((NC * N_PAD,), jnp.float32),
    mesh=_mesh(),
    scratch_types=[
        pltpu.VMEM_SHARED((N_PAD,), jnp.float32),
        pltpu.VMEM((CHT, C), jnp.int32),
        pltpu.VMEM((CHT, C), jnp.int32),
        pltpu.VMEM((C,), jnp.float32),
        pltpu.VMEM((RPT,), jnp.float32),
    ] + [pltpu.SemaphoreType.DMA] * 8,
)
def _deg(src_hbm, dst_hbm, out_hbm, acc, didx, sidx, ones_v, zbuf, *sems):
    cid = lax.axis_index("c")
    sid = lax.axis_index("s")
    for i in range(C // 16):
        ones_v[pl.ds(i * 16, 16)] = jnp.full((16,), 1.0, jnp.float32)
    for i in range(RPT // 16):
        zbuf[pl.ds(i * 16, 16)] = jnp.zeros((16,), jnp.float32)
    base = (cid * NS + sid) * EPW
    _stage_idx(src_hbm, dst_hbm, sidx, didx, base, sems[0])
    pltpu.sync_copy(zbuf, acc.at[pl.ds(sid * RPT, RPT)])
    plsc.subcore_barrier()

    @pl.loop(0, CHT // 8)
    def _(r):
        c0 = r * 8
        descs = [
            pltpu.async_copy(ones_v, acc.at[didx.at[c0 + b]], sems[b], add=True)
            for b in range(8)
        ]
        for d in descs:
            d.wait()

    plsc.subcore_barrier()
    pltpu.sync_copy(acc.at[pl.ds(sid * RPT, RPT)],
                    out_hbm.at[pl.ds(cid * N_PAD + sid * RPT, RPT)])


def _prop_loop(g_hbm, acc, sidx, didx, rows, gsems, ssems, K):
    ROUNDS = CHT // K
    for b in range(K):
        pltpu.async_copy(g_hbm.at[sidx.at[b]], rows.at[b], gsems[b])

    @pl.loop(0, ROUNDS - 1)
    def _(r):
        c0 = r * K
        sds = []
        for b in range(K):
            pltpu.make_async_copy(g_hbm.at[sidx.at[c0 + b]], rows.at[b],
                                  gsems[b]).wait()
            sds.append(pltpu.async_copy(rows.at[b], acc.at[didx.at[c0 + b]],
                                        ssems[b], add=True))
        for b in range(K):
            sds[b].wait()
            pltpu.async_copy(g_hbm.at[sidx.at[c0 + K + b]], rows.at[b],
                             gsems[b])

    c0 = (ROUNDS - 1) * K
    sds = []
    for b in range(K):
        pltpu.make_async_copy(g_hbm.at[sidx.at[c0 + b]], rows.at[b],
                              gsems[b]).wait()
        sds.append(pltpu.async_copy(rows.at[b], acc.at[didx.at[c0 + b]],
                                    ssems[b], add=True))
    for d in sds:
        d.wait()


_K64 = 8
_K128 = 2


@functools.partial(
    pl.kernel,
    out_type=jax.ShapeDtypeStruct((NC * N_PAD, 64), jnp.float32),
    mesh=_mesh(),
    scratch_types=[
        pltpu.VMEM_SHARED((N_PAD, 64), jnp.float32),
        pltpu.VMEM((CHT, C), jnp.int32),
        pltpu.VMEM((CHT, C), jnp.int32),
        pltpu.VMEM((_K64, C, 64), jnp.float32),
        pltpu.VMEM((ZR, 64), jnp.float32),
    ] + [pltpu.SemaphoreType.DMA] * (2 * _K64),
    compiler_params=_SC_LINEAR,
)
def _prop64(g_hbm, src_hbm, dst_hbm, out_hbm, acc, sidx, didx, rows, zbuf,
            *sems):
    K = _K64
    cid = lax.axis_index("c")
    sid = lax.axis_index("s")

    @pl.loop(0, ZR)
    def _(r):
        for q in range(4):
            zbuf[r, pl.ds(q * 16, 16)] = jnp.zeros((16,), jnp.float32)

    base = (cid * NS + sid) * EPW
    _stage_idx(src_hbm, dst_hbm, sidx, didx, base, sems[0])
    zd = [
        pltpu.async_copy(zbuf, acc.at[pl.ds(sid * RPT + z * ZR, ZR)], sems[z])
        for z in range(RPT // ZR)
    ]
    for d in zd:
        d.wait()
    plsc.subcore_barrier()

    _prop_loop(g_hbm, acc, sidx, didx, rows, sems[:K], sems[K:], K)

    plsc.subcore_barrier()
    pltpu.sync_copy(acc.at[pl.ds(sid * RPT, RPT)],
                    out_hbm.at[pl.ds(cid * N_PAD + sid * RPT, RPT)])


@functools.partial(
    pl.kernel,
    out_type=jax.ShapeDtypeStruct((NC * N_PAD, 128), jnp.float32),
    mesh=_mesh(),
    scratch_types=[
        pltpu.VMEM_SHARED((N_PAD, 128), jnp.float32),
        pltpu.VMEM((CHT, C), jnp.int32),
        pltpu.VMEM((CHT, C), jnp.int32),
        pltpu.VMEM((_K128, C, 128), jnp.float32),
    ] + [pltpu.SemaphoreType.DMA] * (2 * _K128 + 1),
)
def _prop128(g_hbm, src_hbm, dst_hbm, out_hbm, acc, sidx, didx, rows, *sems):
    K = _K128
    gsems, ssems, zsem = sems[:K], sems[K:2 * K], sems[2 * K]
    cid = lax.axis_index("c")
    sid = lax.axis_index("s")

    @pl.loop(0, C)
    def _(r):
        for q in range(8):
            rows[0, r, pl.ds(q * 16, 16)] = jnp.zeros((16,), jnp.float32)

    base = (cid * NS + sid) * EPW
    _stage_idx(src_hbm, dst_hbm, sidx, didx, base, zsem)
    zd = [
        pltpu.async_copy(rows.at[0], acc.at[pl.ds(sid * RPT + z * C, C)], zsem)
        for z in range(RPT // C)
    ]
    for d in zd:
        d.wait()
    plsc.subcore_barrier()

    _prop_loop(g_hbm, acc, sidx, didx, rows, gsems, ssems, K)

    plsc.subcore_barrier()
    pltpu.sync_copy(acc.at[pl.ds(sid * RPT, RPT)],
                    out_hbm.at[pl.ds(cid * N_PAD + sid * RPT, RPT)])


R = 1024
GRID = N_PAD // R


def _tc_first(x, W, c0, c1):
    Din, Dout = W.shape

    def body(x_ref, w_ref, c0_ref, c1_ref, g_ref, dinv_ref):
        h = jnp.dot(x_ref[...], w_ref[...], preferred_element_type=jnp.float32)
        dinv = lax.rsqrt(c0_ref[...] + c1_ref[...] + 1.0)
        dinv_ref[...] = dinv
        g_ref[...] = h * dinv

    return pl.pallas_call(
        body,
        grid=(GRID,),
        in_specs=[
            pl.BlockSpec((R, Din), lambda i: (i, 0)),
            pl.BlockSpec((Din, Dout), lambda i: (0, 0)),
            pl.BlockSpec((R, 1), lambda i: (i, 0)),
            pl.BlockSpec((R, 1), lambda i: (i, 0)),
        ],
        out_specs=[
            pl.BlockSpec((R, Dout), lambda i: (i, 0)),
            pl.BlockSpec((R, 1), lambda i: (i, 0)),
        ],
        out_shape=[
            jax.ShapeDtypeStruct((N, Dout), jnp.float32),
            jax.ShapeDtypeStruct((N_PAD, 1), jnp.float32),
        ],
    )(x, W, c0, c1)


def _tc_mid(acc, g, dinv, b, W):
    Din, Dout = W.shape

    def body(a0_ref, a1_ref, g_ref, dv_ref, b_ref, w_ref, o_ref):
        dcol = dv_ref[...]
        s = dcol * (a0_ref[...] + a1_ref[...] + g_ref[...]) + b_ref[...]
        act = jnp.where(s >= 0, s, 0.2 * s)
        h = jnp.dot(act, w_ref[...], preferred_element_type=jnp.float32)
        o_ref[...] = h * dcol

    return pl.pallas_call(
        body,
        grid=(GRID,),
        in_specs=[
            pl.BlockSpec((R, Din), lambda i: (i, 0)),
            pl.BlockSpec((R, Din), lambda i: (i + GRID, 0)),
            pl.BlockSpec((R, Din), lambda i: (i, 0)),
            pl.BlockSpec((R, 1), lambda i: (i, 0)),
            pl.BlockSpec((1, Din), lambda i: (0, 0)),
            pl.BlockSpec((Din, Dout), lambda i: (0, 0)),
        ],
        out_specs=pl.BlockSpec((R, Dout), lambda i: (i, 0)),
        out_shape=jax.ShapeDtypeStruct((N, Dout), jnp.float32),
    )(acc, acc, g, dinv, b, W)


def _tc_last(acc, g, dinv, b):
    F = g.shape[1]

    def body(a0_ref, a1_ref, g_ref, dv_ref, b_ref, o_ref):
        o_ref[...] = (dv_ref[...] * (a0_ref[...] + a1_ref[...] + g_ref[...])
                      + b_ref[...])

    return pl.pallas_call(
        body,
        grid=(GRID,),
        in_specs=[
            pl.BlockSpec((R, F), lambda i: (i, 0)),
            pl.BlockSpec((R, F), lambda i: (i + GRID, 0)),
            pl.BlockSpec((R, F), lambda i: (i, 0)),
            pl.BlockSpec((R, 1), lambda i: (i, 0)),
            pl.BlockSpec((1, F), lambda i: (0, 0)),
        ],
        out_specs=pl.BlockSpec((R, F), lambda i: (i, 0)),
        out_shape=jax.ShapeDtypeStruct((N, F), jnp.float32),
    )(acc, acc, g, dinv, b)


def kernel(x, edge_index, W1, b1, W2, b2, W3, b3):
    ei = edge_index.astype(jnp.int32)
    src, dst = ei[0], ei[1]

    cnt = _deg(src, dst)
    c0 = cnt[:N_PAD].reshape(N_PAD, 1)
    c1 = cnt[N_PAD:].reshape(N_PAD, 1)

    g1, dinv = _tc_first(x, W1, c0, c1)
    acc = _prop128(g1, src, dst)
    g2 = _tc_mid(acc, g1, dinv, b1.reshape(1, -1), W2)
    acc = _prop64(g2, src, dst)
    g3 = _tc_mid(acc, g2, dinv, b2.reshape(1, -1), W3)
    acc = _prop64(g3, src, dst)
    return _tc_last(acc, g3, dinv, b3.reshape(1, -1))
